# Initial kernel scaffold; baseline (speedup 1.0000x reference)
import jax
import jax.numpy as jnp
from jax.experimental import pallas as pl


def kernel(input, emb, W1, b1, W2, b2):
    raise NotImplementedError("write your pallas kernel here")


# speedup vs baseline: 30.9470x; 30.9470x over previous
import functools

import jax
from jax import lax
import jax.numpy as jnp
from jax.experimental import pallas as pl
from jax.experimental.pallas import tpu as pltpu
from jax.experimental.pallas import tpu_sc as plsc

_B = 16384
_SEQ = 50
_EMB = 32
_HID = 100
_L = 16
_NC = 2
_NS = 16
_NW = _NC * _NS
_BPW = _B // _NW
_CH = 32
_NIT = _BPW // _CH
_TOK = _CH * _SEQ
_G = 80
_NG = _TOK // _G


def _fold_body(w1_ref, w2_ref, b1_ref, b2_ref, w_ref, b_ref):
    w2 = w2_ref[...]
    w_ref[...] = jnp.sum(w1_ref[...] * w2, axis=1, keepdims=True)
    b_ref[...] = jnp.sum(b1_ref[...] * w2, axis=1, keepdims=True) + b2_ref[...]


_fold = pl.pallas_call(
    _fold_body,
    out_shape=[
        jax.ShapeDtypeStruct((_SEQ * _EMB, 1), jnp.float32),
        jax.ShapeDtypeStruct((1, 1), jnp.float32),
    ],
)


def _sc_body(emb_hbm, idx_hbm, w_hbm, out_hbm, idx_v, rows_v, w_v, out_v, sem):
    wid = lax.axis_index("s") * _NC + lax.axis_index("c")
    pltpu.sync_copy(w_hbm, w_v)
    base_b = wid * _BPW

    @pl.loop(0, _NIT)
    def _chunk(it):
        b0 = base_b + it * _CH
        pltpu.sync_copy(idx_hbm.at[pl.ds(b0 * _SEQ, _TOK)], idx_v)
        copies = [
            pltpu.async_copy(
                emb_hbm.at[idx_v.at[pl.ds(j * _G, _G)]],
                rows_v.at[pl.ds(j * _G, _G)],
                sem,
            )
            for j in range(_NG)
        ]
        for cp in copies:
            cp.wait()

        @pl.loop(0, _CH)
        def _row(bb):
            def sbody(s, carry):
                a0, a1 = carry
                r = bb * _SEQ + s
                a0 = a0 + rows_v[r, pl.ds(0, _L)] * w_v[s, pl.ds(0, _L)]
                a1 = a1 + rows_v[r, pl.ds(_L, _L)] * w_v[s, pl.ds(_L, _L)]
                return (a0, a1)

            z = jnp.zeros((_L,), jnp.float32)
            a0, a1 = lax.fori_loop(0, _SEQ, sbody, (z, z))
            out_v[bb, :] = a0 + a1

        pltpu.sync_copy(out_v, out_hbm.at[pl.ds(b0, _CH)])


@functools.cache
def _sc_gather_reduce():
    return pl.kernel(
        _sc_body,
        out_type=jax.ShapeDtypeStruct((_B, _L), jnp.float32),
        mesh=plsc.VectorSubcoreMesh(core_axis_name="c", subcore_axis_name="s"),
        scratch_types=[
            pltpu.VMEM((_TOK,), jnp.int32),
            pltpu.VMEM((_TOK, _EMB), jnp.float32),
            pltpu.VMEM((_SEQ, _EMB), jnp.float32),
            pltpu.VMEM((_CH, _L), jnp.float32),
            pltpu.SemaphoreType.DMA,
        ],
        compiler_params=pltpu.CompilerParams(use_tc_tiling_on_sc=False),
    )


def _fin_body(x_ref, b_ref, o_ref):
    s = jnp.sum(x_ref[...], axis=1, keepdims=True) + b_ref[0, 0]
    o_ref[...] = jax.nn.sigmoid(s)


_finish = pl.pallas_call(
    _fin_body,
    out_shape=jax.ShapeDtypeStruct((_B, 1), jnp.float32),
)


def kernel(input, emb, W1, b1, W2, b2):
    idx = input.reshape(-1).astype(jnp.int32)
    w_flat, bscal = _fold(
        W1,
        W2.reshape(1, _HID),
        b1.reshape(1, _HID),
        b2.reshape(1, 1),
    )
    w50 = w_flat.reshape(_SEQ, _EMB)
    out32 = _sc_gather_reduce()(emb, idx, w50)
    return _finish(out32, bscal)


# speedup vs baseline: 33.5342x; 1.0836x over previous
import functools

import jax
from jax import lax
import jax.numpy as jnp
from jax.experimental import pallas as pl
from jax.experimental.pallas import tpu as pltpu
from jax.experimental.pallas import tpu_sc as plsc

_B = 16384
_SEQ = 50
_EMB = 32
_HID = 100
_L = 16
_NC = 2
_NS = 16
_NW = _NC * _NS
_BPW = _B // _NW
_CH = 32
_NIT = _BPW // _CH
_TOK = _CH * _SEQ
_G = 80
_NG = _TOK // _G


def _fold_body(w1_ref, w2_ref, b1_ref, b2_ref, w_ref, b_ref):
    w2 = w2_ref[...]
    w_ref[...] = jnp.sum(w1_ref[...] * w2, axis=1, keepdims=True)
    b_ref[...] = jnp.sum(b1_ref[...] * w2, axis=1, keepdims=True) + b2_ref[...]


_fold = pl.pallas_call(
    _fold_body,
    out_shape=[
        jax.ShapeDtypeStruct((_SEQ * _EMB, 1), jnp.float32),
        jax.ShapeDtypeStruct((1, 1), jnp.float32),
    ],
)


_VOCAB = 1000000
_TTB = 2048
_NTB = (_VOCAB + _TTB - 1) // _TTB
_VPAD = _NTB * _TTB


def _tr_body(xt_ref, o_ref):
    q = _TTB // 4
    for c in range(4):
        o_ref[:, 32 * c:32 * (c + 1)] = jnp.swapaxes(
            xt_ref[:, q * c:q * (c + 1)], 0, 1)


_transpose_table = pl.pallas_call(
    _tr_body,
    grid=(_NTB,),
    in_specs=[pl.BlockSpec((_EMB, _TTB), lambda j: (0, j))],
    out_specs=pl.BlockSpec((_TTB // 4, 128), lambda j: (j, 0)),
    out_shape=jax.ShapeDtypeStruct((_VPAD // 4, 128), jnp.float32),
)


def _sc_body(emb_hbm, idx_hbm, w_hbm, out_hbm, idx_v, rows_v, w_v, out_v, sem):
    wid = lax.axis_index("s") * _NC + lax.axis_index("c")
    pltpu.sync_copy(w_hbm, w_v)
    base_b = wid * _BPW

    @pl.loop(0, _NIT)
    def _chunk(it):
        b0 = base_b + it * _CH
        pltpu.sync_copy(idx_hbm.at[pl.ds(b0 * _SEQ, _TOK)], idx_v)

        @pl.loop(0, _TOK // _L)
        def _perm(k):
            v = idx_v[pl.ds(k * _L, _L)]
            idx_v[pl.ds(k * _L, _L)] = (
                (v & -2048) + ((v & 511) << 2) + ((v >> 9) & 3))

        copies = [
            pltpu.async_copy(
                emb_hbm.at[idx_v.at[pl.ds(j * _G, _G)]],
                rows_v.at[pl.ds(j * _G, _G)],
                sem,
            )
            for j in range(_NG)
        ]
        for cp in copies:
            cp.wait()

        @pl.loop(0, _CH)
        def _row(bb):
            def sbody(s, carry):
                a0, a1 = carry
                r = bb * _SEQ + s
                a0 = a0 + rows_v[r, pl.ds(0, _L)] * w_v[s, pl.ds(0, _L)]
                a1 = a1 + rows_v[r, pl.ds(_L, _L)] * w_v[s, pl.ds(_L, _L)]
                return (a0, a1)

            z = jnp.zeros((_L,), jnp.float32)
            a0, a1 = lax.fori_loop(0, _SEQ, sbody, (z, z))
            out_v[bb, :] = a0 + a1

        pltpu.sync_copy(out_v, out_hbm.at[pl.ds(b0, _CH)])


@functools.cache
def _sc_gather_reduce():
    return pl.kernel(
        _sc_body,
        out_type=jax.ShapeDtypeStruct((_B, _L), jnp.float32),
        mesh=plsc.VectorSubcoreMesh(core_axis_name="c", subcore_axis_name="s"),
        scratch_types=[
            pltpu.VMEM((_TOK,), jnp.int32),
            pltpu.VMEM((_TOK, _EMB), jnp.float32),
            pltpu.VMEM((_SEQ, _EMB), jnp.float32),
            pltpu.VMEM((_CH, _L), jnp.float32),
            pltpu.SemaphoreType.DMA,
        ],
        compiler_params=pltpu.CompilerParams(use_tc_tiling_on_sc=False),
    )


def _fin_body(x_ref, b_ref, o_ref):
    s = jnp.sum(x_ref[...], axis=1, keepdims=True) + b_ref[0, 0]
    o_ref[...] = jax.nn.sigmoid(s)


_finish = pl.pallas_call(
    _fin_body,
    out_shape=jax.ShapeDtypeStruct((_B, 1), jnp.float32),
)


def kernel(input, emb, W1, b1, W2, b2):
    idx = input.reshape(-1).astype(jnp.int32)
    w_flat, bscal = _fold(
        W1,
        W2.reshape(1, _HID),
        b1.reshape(1, _HID),
        b2.reshape(1, 1),
    )
    w50 = w_flat.reshape(_SEQ, _EMB)
    emb_rm = _transpose_table(emb.T).reshape(_VPAD, _EMB)
    out32 = _sc_gather_reduce()(emb_rm, idx, w50)
    return _finish(out32, bscal)
